# Initial kernel scaffold; baseline (speedup 1.0000x reference)
import jax
import jax.numpy as jnp
from jax.experimental import pallas as pl


def kernel(x, edge_index, edge_attr, batch, W_src, b_src, W_dst, b_dst, W_edge, b_edge, W1, b1, g1, be1, W2, b2, g2, be2, W3, b3, Wa, ba, Wb, bb):
    raise NotImplementedError("write your pallas kernel here")


# speedup vs baseline: 1.9605x; 1.9605x over previous
import functools
import numpy as np
import jax
import jax.numpy as jnp
from jax.experimental import pallas as pl

N = 10000
E = 320000
D_FEAT = 128
D_EDGE = 16
D_HID = 300
D_MLP = 600
D_LIN = 256
D_OUT = 128
NUM_GRAPHS = 64
EPS = 1e-7
BN_EPS = 1e-5

NODE_BLK = 200
EDGE_BLK = 1000
MLP_BLK = 200


def _node_proj_body(x_ref, ws_ref, bs_ref, wd_ref, bd_ref, xs_ref, xd_ref):
    xb = x_ref[...]
    xs_ref[...] = jnp.dot(xb, ws_ref[...], preferred_element_type=jnp.float32) + bs_ref[...]
    xd_ref[...] = jnp.dot(xb, wd_ref[...], preferred_element_type=jnp.float32) + bd_ref[...]


def _edge_proj_body(ea_ref, we_ref, be_ref, e_ref):
    e_ref[...] = jnp.dot(ea_ref[...], we_ref[...], preferred_element_type=jnp.float32) + be_ref[...]


def _mlp_pool_body(h_ref, batch_ref, w1_ref, b1_ref, w2_ref, b2_ref, w3_ref, b3_ref,
                   wa_ref, ba_ref, wb_ref, bb_ref, out_ref, pool_acc, cnt_acc):
    i = pl.program_id(0)

    @pl.when(i == 0)
    def _init():
        pool_acc[...] = jnp.zeros_like(pool_acc)
        cnt_acc[...] = jnp.zeros_like(cnt_acc)

    h = h_ref[...]
    h = jnp.maximum(jnp.dot(h, w1_ref[...], preferred_element_type=jnp.float32) + b1_ref[...], 0.0)
    h = jnp.maximum(jnp.dot(h, w2_ref[...], preferred_element_type=jnp.float32) + b2_ref[...], 0.0)
    h = jnp.dot(h, w3_ref[...], preferred_element_type=jnp.float32) + b3_ref[...]
    h = jnp.maximum(jnp.dot(h, wa_ref[...], preferred_element_type=jnp.float32) + ba_ref[...], 0.0)
    h = jnp.dot(h, wb_ref[...], preferred_element_type=jnp.float32) + bb_ref[...]

    brow = batch_ref[0]
    gids = jax.lax.broadcasted_iota(jnp.int32, (NUM_GRAPHS, MLP_BLK), 0)
    oh = (gids == brow).astype(jnp.float32)
    pool_acc[...] += jnp.dot(oh, h, preferred_element_type=jnp.float32)
    cnt = jnp.sum(oh, axis=1, keepdims=True)
    cnt_acc[...] += jnp.broadcast_to(cnt, cnt_acc.shape)

    @pl.when(i == pl.num_programs(0) - 1)
    def _fin():
        out_ref[...] = pool_acc[...] / jnp.maximum(cnt_acc[...], 1.0)


def _full(shape):
    nd = len(shape)
    return pl.BlockSpec(shape, lambda i: (0,) * nd)


def _node_proj(x, W_src, b_src, W_dst, b_dst):
    grid = (N // NODE_BLK,)
    return pl.pallas_call(
        _node_proj_body,
        grid=grid,
        in_specs=[
            pl.BlockSpec((NODE_BLK, D_FEAT), lambda i: (i, 0)),
            _full((D_FEAT, D_HID)),
            _full((1, D_HID)),
            _full((D_FEAT, D_HID)),
            _full((1, D_HID)),
        ],
        out_specs=[
            pl.BlockSpec((NODE_BLK, D_HID), lambda i: (i, 0)),
            pl.BlockSpec((NODE_BLK, D_HID), lambda i: (i, 0)),
        ],
        out_shape=[
            jax.ShapeDtypeStruct((N, D_HID), jnp.float32),
            jax.ShapeDtypeStruct((N, D_HID), jnp.float32),
        ],
    )(x, W_src, b_src.reshape(1, -1), W_dst, b_dst.reshape(1, -1))


def _edge_proj(edge_attr, W_edge, b_edge):
    grid = (E // EDGE_BLK,)
    return pl.pallas_call(
        _edge_proj_body,
        grid=grid,
        in_specs=[
            pl.BlockSpec((EDGE_BLK, D_EDGE), lambda i: (i, 0)),
            _full((D_EDGE, D_HID)),
            _full((1, D_HID)),
        ],
        out_specs=pl.BlockSpec((EDGE_BLK, D_HID), lambda i: (i, 0)),
        out_shape=jax.ShapeDtypeStruct((E, D_HID), jnp.float32),
    )(edge_attr, W_edge, b_edge.reshape(1, -1))


def _mlp_pool(h, batch, W1f, b1f, W2f, b2f, W3, b3, Wa, ba, Wb, bb):
    grid = (N // MLP_BLK,)
    batch3 = batch.reshape(N // MLP_BLK, 1, MLP_BLK)
    return pl.pallas_call(
        _mlp_pool_body,
        grid=grid,
        in_specs=[
            pl.BlockSpec((MLP_BLK, D_HID), lambda i: (i, 0)),
            pl.BlockSpec((1, 1, MLP_BLK), lambda i: (i, 0, 0)),
            _full((D_HID, D_MLP)), _full((1, D_MLP)),
            _full((D_MLP, D_MLP)), _full((1, D_MLP)),
            _full((D_MLP, D_HID)), _full((1, D_HID)),
            _full((D_HID, D_LIN)), _full((1, D_LIN)),
            _full((D_LIN, D_OUT)), _full((1, D_OUT)),
        ],
        out_specs=_full((NUM_GRAPHS, D_OUT)),
        out_shape=jax.ShapeDtypeStruct((NUM_GRAPHS, D_OUT), jnp.float32),
        scratch_shapes=[
            pltpu_scratch((NUM_GRAPHS, D_OUT)),
            pltpu_scratch((NUM_GRAPHS, 128)),
        ],
    )(h, batch3, W1f, b1f.reshape(1, -1), W2f, b2f.reshape(1, -1),
      W3, b3.reshape(1, -1), Wa, ba.reshape(1, -1), Wb, bb.reshape(1, -1))


from jax.experimental.pallas import tpu as pltpu


def pltpu_scratch(shape):
    return pltpu.VMEM(shape, jnp.float32)


def kernel(x, edge_index, edge_attr, batch, W_src, b_src, W_dst, b_dst, W_edge, b_edge,
           W1, b1, g1, be1, W2, b2, g2, be2, W3, b3, Wa, ba, Wb, bb):
    inv = np.float32(1.0 / np.sqrt(1.0 + BN_EPS))
    s1 = g1 * inv
    W1f = W1 * s1[None, :]
    b1f = b1 * s1 + be1
    s2 = g2 * inv
    W2f = W2 * s2[None, :]
    b2f = b2 * s2 + be2

    xs, xd = _node_proj(x, W_src, b_src, W_dst, b_dst)
    e = _edge_proj(edge_attr, W_edge, b_edge)

    src = edge_index[0]
    dst = edge_index[1]
    msg = jnp.maximum(xs[src] + e, 0.0) + EPS
    p = jnp.exp(msg)
    s = jax.ops.segment_sum(p, dst, num_segments=N)
    w = jax.ops.segment_sum(p * msg, dst, num_segments=N)
    agg = jnp.where(s > 0.0, w / jnp.where(s > 0.0, s, 1.0), 0.0)
    out0 = agg + xd

    return _mlp_pool(out0, batch, W1f, b1f, W2f, b2f, W3, b3, Wa, ba, Wb, bb)


# speedup vs baseline: 1.9606x; 1.0000x over previous
import numpy as np
import jax
import jax.numpy as jnp
from jax.experimental import pallas as pl
from jax.experimental.pallas import tpu as pltpu

N = 10000
E = 320000
D_FEAT = 128
D_EDGE = 16
D_HID = 300
D_MLP = 600
D_LIN = 256
D_OUT = 128
NUM_GRAPHS = 64
EPS = 1e-7
BN_EPS = 1e-5

NODE_BLK = 200
EDGE_BLK = 1000
MLP_BLK = 200


def _node_proj_body(x_ref, ws_ref, bs_ref, wd_ref, bd_ref, xs_ref, xd_ref):
    xb = x_ref[...]
    xs_ref[...] = jnp.dot(xb, ws_ref[...], preferred_element_type=jnp.float32) + bs_ref[...]
    xd_ref[...] = jnp.dot(xb, wd_ref[...], preferred_element_type=jnp.float32) + bd_ref[...]


def _edge_proj_body(ea_ref, we_ref, be_ref, e_ref):
    e_ref[...] = jnp.dot(ea_ref[...], we_ref[...], preferred_element_type=jnp.float32) + be_ref[...]


def _mlp_pool_body(h_ref, batch_ref, w1_ref, b1_ref, w2_ref, b2_ref, w3_ref, b3_ref,
                   wa_ref, ba_ref, wb_ref, bb_ref, out_ref, pool_acc, cnt_acc):
    i = pl.program_id(0)

    @pl.when(i == 0)
    def _init():
        pool_acc[...] = jnp.zeros_like(pool_acc)
        cnt_acc[...] = jnp.zeros_like(cnt_acc)

    h = h_ref[...]
    h = jnp.maximum(jnp.dot(h, w1_ref[...], preferred_element_type=jnp.float32) + b1_ref[...], 0.0)
    h = jnp.maximum(jnp.dot(h, w2_ref[...], preferred_element_type=jnp.float32) + b2_ref[...], 0.0)
    h = jnp.dot(h, w3_ref[...], preferred_element_type=jnp.float32) + b3_ref[...]
    h = jnp.maximum(jnp.dot(h, wa_ref[...], preferred_element_type=jnp.float32) + ba_ref[...], 0.0)
    h = jnp.dot(h, wb_ref[...], preferred_element_type=jnp.float32) + bb_ref[...]

    brow = batch_ref[0]
    gids = jax.lax.broadcasted_iota(jnp.int32, (NUM_GRAPHS, MLP_BLK), 0)
    oh = (gids == brow).astype(jnp.float32)
    pool_acc[...] += jnp.dot(oh, h, preferred_element_type=jnp.float32)
    cnt = jnp.sum(oh, axis=1, keepdims=True)
    cnt_acc[...] += jnp.broadcast_to(cnt, cnt_acc.shape)

    @pl.when(i == pl.num_programs(0) - 1)
    def _fin():
        out_ref[...] = pool_acc[...] / jnp.maximum(cnt_acc[...], 1.0)


def _full(shape):
    nd = len(shape)
    return pl.BlockSpec(shape, lambda *args: (0,) * nd)


def _node_proj(x, W_src, b_src, W_dst, b_dst):
    grid = (N // NODE_BLK,)
    return pl.pallas_call(
        _node_proj_body,
        grid=grid,
        in_specs=[
            pl.BlockSpec((NODE_BLK, D_FEAT), lambda i: (i, 0)),
            _full((D_FEAT, D_HID)),
            _full((1, D_HID)),
            _full((D_FEAT, D_HID)),
            _full((1, D_HID)),
        ],
        out_specs=[
            pl.BlockSpec((NODE_BLK, D_HID), lambda i: (i, 0)),
            pl.BlockSpec((NODE_BLK, D_HID), lambda i: (i, 0)),
        ],
        out_shape=[
            jax.ShapeDtypeStruct((N, D_HID), jnp.float32),
            jax.ShapeDtypeStruct((N, D_HID), jnp.float32),
        ],
    )(x, W_src, b_src.reshape(1, -1), W_dst, b_dst.reshape(1, -1))


def _edge_proj(edge_attr, W_edge, b_edge):
    grid = (E // EDGE_BLK,)
    return pl.pallas_call(
        _edge_proj_body,
        grid=grid,
        in_specs=[
            pl.BlockSpec((EDGE_BLK, D_EDGE), lambda i: (i, 0)),
            _full((D_EDGE, D_HID)),
            _full((1, D_HID)),
        ],
        out_specs=pl.BlockSpec((EDGE_BLK, D_HID), lambda i: (i, 0)),
        out_shape=jax.ShapeDtypeStruct((E, D_HID), jnp.float32),
    )(edge_attr, W_edge, b_edge.reshape(1, -1))


def _mlp_pool(h, batch, W1f, b1f, W2f, b2f, W3, b3, Wa, ba, Wb, bb):
    grid = (N // MLP_BLK,)
    batch3 = batch.reshape(N // MLP_BLK, 1, MLP_BLK)
    return pl.pallas_call(
        _mlp_pool_body,
        grid=grid,
        in_specs=[
            pl.BlockSpec((MLP_BLK, D_HID), lambda i: (i, 0)),
            pl.BlockSpec((1, 1, MLP_BLK), lambda i: (i, 0, 0)),
            _full((D_HID, D_MLP)), _full((1, D_MLP)),
            _full((D_MLP, D_MLP)), _full((1, D_MLP)),
            _full((D_MLP, D_HID)), _full((1, D_HID)),
            _full((D_HID, D_LIN)), _full((1, D_LIN)),
            _full((D_LIN, D_OUT)), _full((1, D_OUT)),
        ],
        out_specs=_full((NUM_GRAPHS, D_OUT)),
        out_shape=jax.ShapeDtypeStruct((NUM_GRAPHS, D_OUT), jnp.float32),
        scratch_shapes=[
            pltpu.VMEM((NUM_GRAPHS, D_OUT), jnp.float32),
            pltpu.VMEM((NUM_GRAPHS, 128), jnp.float32),
        ],
    )(h, batch3, W1f, b1f.reshape(1, -1), W2f, b2f.reshape(1, -1),
      W3, b3.reshape(1, -1), Wa, ba.reshape(1, -1), Wb, bb.reshape(1, -1))


def kernel(x, edge_index, edge_attr, batch, W_src, b_src, W_dst, b_dst, W_edge, b_edge,
           W1, b1, g1, be1, W2, b2, g2, be2, W3, b3, Wa, ba, Wb, bb):
    inv = np.float32(1.0 / np.sqrt(1.0 + BN_EPS))
    s1 = g1 * inv
    W1f = W1 * s1[None, :]
    b1f = b1 * s1 + be1
    s2 = g2 * inv
    W2f = W2 * s2[None, :]
    b2f = b2 * s2 + be2

    xs, xd = _node_proj(x, W_src, b_src, W_dst, b_dst)
    e = _edge_proj(edge_attr, W_edge, b_edge)

    src = edge_index[0]
    dst = edge_index[1]
    msg = jnp.maximum(xs[src] + e, 0.0) + EPS
    p = jnp.exp(msg)
    s = jax.ops.segment_sum(p, dst, num_segments=N)
    w = jax.ops.segment_sum(p * msg, dst, num_segments=N)
    agg = jnp.where(s > 0.0, w / jnp.where(s > 0.0, s, 1.0), 0.0)
    out0 = agg + xd

    return _mlp_pool(out0, batch, W1f, b1f, W2f, b2f, W3, b3, Wa, ba, Wb, bb)
